# Initial kernel scaffold; baseline (speedup 1.0000x reference)
import jax
import jax.numpy as jnp
from jax.experimental import pallas as pl


def kernel(origin_fmap, text, emb_W, emb_b, W_ih0, W_hh0, b_ih0, b_hh0, W_ih1, W_hh1, b_ih1, b_hh1, gen_W, gen_b, is_train):
    raise NotImplementedError("write your pallas kernel here")


# speedup vs baseline: 1.1437x; 1.1437x over previous
import functools

import jax
import jax.numpy as jnp
from jax.experimental import pallas as pl
from jax.experimental.pallas import tpu as pltpu

F32 = jnp.float32


def _decode_kernel(
    fmap_any,
    w_any,
    onehot_ref,
    emb_ref,
    b0_ref,
    b1_ref,
    genw_ref,
    genb_ref,
    g_ref,
    ctx_ref,
    mask_ref,
    fmap_v,
    w_v,
    h0_v, c0_v, h1_v, c1_v,
    sem0, sem1,
    *, bc, num_steps, hid,
):
    i = pl.program_id(0)
    cp_f = pltpu.make_async_copy(fmap_any.at[pl.ds(i * bc, bc)], fmap_v, sem0)
    cp_f.start()
    cp_w = pltpu.make_async_copy(w_any, w_v, sem1)
    cp_w.start()

    zeros = jnp.zeros((bc, hid), F32)
    h0_v[...] = zeros
    c0_v[...] = zeros
    h1_v[...] = zeros
    c1_v[...] = zeros

    cp_w.wait()
    cp_f.wait()

    def lstm(x, h, c, w_ih, w_hh, b):
        gates = (jnp.dot(x, w_ih, preferred_element_type=F32)
                 + jnp.dot(h, w_hh, preferred_element_type=F32) + b)
        gi = jax.nn.sigmoid(gates[:, 0:hid])
        gf = jax.nn.sigmoid(gates[:, hid:2 * hid])
        gg = jnp.tanh(gates[:, 2 * hid:3 * hid])
        go = jax.nn.sigmoid(gates[:, 3 * hid:4 * hid])
        c_new = gf * c + gi * gg
        h_new = go * jnp.tanh(c_new)
        return h_new, c_new

    def step(t, _):
        oh = onehot_ref[t]
        x = jnp.dot(oh, emb_ref[...], preferred_element_type=F32)
        h0, c0 = lstm(x, h0_v[...], c0_v[...], w_v[0], w_v[1], b0_ref[...])
        h0_v[...] = h0
        c0_v[...] = c0
        h1, c1 = lstm(h0, h1_v[...], c1_v[...], w_v[2], w_v[3], b1_ref[...])
        h1_v[...] = h1
        c1_v[...] = c1

        fm = fmap_v[...]
        logits = jnp.sum(fm * h1[:, None, :], axis=-1)
        a = jax.nn.sigmoid(logits)
        mask_ref[t] = a
        ctx = jnp.sum(fm * a[:, :, None], axis=1)
        ctx_ref[t] = ctx
        g_ref[t] = (jnp.dot(ctx, genw_ref[...], preferred_element_type=F32)
                    + genb_ref[...])
        return ()

    jax.lax.fori_loop(0, num_steps, step, (), unroll=False)


def kernel(origin_fmap, text, emb_W, emb_b, W_ih0, W_hh0, b_ih0, b_hh0,
           W_ih1, W_hh1, b_ih1, b_hh1, gen_W, gen_b, is_train):
    b, c, h, w = origin_fmap.shape
    hid = c
    l = h * w
    t = text.shape[1]
    nc = gen_W.shape[0]
    ncp = 128
    bc = 32
    nb = b // bc

    fmap = jnp.transpose(origin_fmap, (0, 2, 3, 1)).reshape(b, l, hid)
    onehot = (text.T[:, :, None] == jnp.arange(ncp)[None, None, :]).astype(F32)
    emb_pad = jnp.zeros((ncp, hid), F32).at[:nc].set(emb_W.T) + emb_b[None, :]
    w_all = jnp.stack([W_ih0.T, W_hh0.T, W_ih1.T, W_hh1.T])
    b0 = (b_ih0 + b_hh0)[None, :]
    b1 = (b_ih1 + b_hh1)[None, :]
    genw = jnp.zeros((hid, ncp), F32).at[:, :nc].set(gen_W.T)
    genb = jnp.zeros((1, ncp), F32).at[0, :nc].set(gen_b)

    kfn = functools.partial(_decode_kernel, bc=bc, num_steps=t, hid=hid)
    g_all, ctx_all, mask_all = pl.pallas_call(
        kfn,
        grid=(nb,),
        in_specs=[
            pl.BlockSpec(memory_space=pl.ANY),
            pl.BlockSpec(memory_space=pl.ANY),
            pl.BlockSpec((t, bc, ncp), lambda i: (0, i, 0)),
            pl.BlockSpec((ncp, hid), lambda i: (0, 0)),
            pl.BlockSpec((1, 4 * hid), lambda i: (0, 0)),
            pl.BlockSpec((1, 4 * hid), lambda i: (0, 0)),
            pl.BlockSpec((hid, ncp), lambda i: (0, 0)),
            pl.BlockSpec((1, ncp), lambda i: (0, 0)),
        ],
        out_specs=[
            pl.BlockSpec((t, bc, ncp), lambda i: (0, i, 0)),
            pl.BlockSpec((t, bc, hid), lambda i: (0, i, 0)),
            pl.BlockSpec((t, bc, l), lambda i: (0, i, 0)),
        ],
        out_shape=[
            jax.ShapeDtypeStruct((t, b, ncp), F32),
            jax.ShapeDtypeStruct((t, b, hid), F32),
            jax.ShapeDtypeStruct((t, b, l), F32),
        ],
        scratch_shapes=[
            pltpu.VMEM((bc, l, hid), F32),
            pltpu.VMEM((4, hid, 4 * hid), F32),
            pltpu.VMEM((bc, hid), F32),
            pltpu.VMEM((bc, hid), F32),
            pltpu.VMEM((bc, hid), F32),
            pltpu.VMEM((bc, hid), F32),
            pltpu.SemaphoreType.DMA,
            pltpu.SemaphoreType.DMA,
        ],
        compiler_params=pltpu.CompilerParams(
            dimension_semantics=("parallel",),
            vmem_limit_bytes=56 * 1024 * 1024,
        ),
        name="hybrid_branch_decode",
    )(fmap, w_all, onehot, emb_pad, b0, b1, genw, genb)

    g = jnp.transpose(g_all, (1, 0, 2))[:, :, :nc]
    output_hiddens = jnp.transpose(ctx_all, (1, 0, 2))
    masks = jnp.transpose(mask_all, (1, 0, 2)).reshape(b, t, h, w)
    return g, output_hiddens, masks


# speedup vs baseline: 1.7997x; 1.5735x over previous
import functools

import jax
import jax.numpy as jnp
from jax.experimental import pallas as pl
from jax.experimental.pallas import tpu as pltpu

F32 = jnp.float32


def _decode_kernel(
    fmap_any,
    w_any,
    onehot_ref,
    emb_ref,
    b0_ref,
    b1_ref,
    genw_ref,
    genb_ref,
    g_ref,
    ctx_ref,
    mask_ref,
    fmap_v,
    w_v,
    h0_v, c0_v, h1_v, c1_v,
    att_v,
    sem0, sem1,
    *, bc, num_steps, hid,
):
    i = pl.program_id(0)
    cp_f = pltpu.make_async_copy(fmap_any.at[pl.ds(i * bc, bc)], fmap_v, sem0)
    cp_f.start()
    cp_w = pltpu.make_async_copy(w_any, w_v, sem1)
    cp_w.start()

    zeros = jnp.zeros((bc, hid), F32)
    h0_v[...] = zeros
    c0_v[...] = zeros
    h1_v[...] = zeros
    c1_v[...] = zeros

    cp_w.wait()
    cp_f.wait()

    bf = jnp.bfloat16

    def lstm(x, h, c, w_ih, w_hh, b):
        gates = (jnp.dot(x.astype(bf), w_ih, preferred_element_type=F32)
                 + jnp.dot(h.astype(bf), w_hh, preferred_element_type=F32) + b)
        gi = jax.nn.sigmoid(gates[:, 0:hid])
        gf = jax.nn.sigmoid(gates[:, hid:2 * hid])
        gg = jnp.tanh(gates[:, 2 * hid:3 * hid])
        go = jax.nn.sigmoid(gates[:, 3 * hid:4 * hid])
        c_new = gf * c + gi * gg
        h_new = go * jnp.tanh(c_new)
        return h_new, c_new

    def lstm_step(t):
        oh = onehot_ref[t]
        x = jnp.dot(oh, emb_ref[...], preferred_element_type=F32)
        h0, c0 = lstm(x, h0_v[...], c0_v[...], w_v[0], w_v[1], b0_ref[...])
        h0_v[...] = h0
        c0_v[...] = c0
        h1, c1 = lstm(h0, h1_v[...], c1_v[...], w_v[2], w_v[3], b1_ref[...])
        h1_v[...] = h1
        c1_v[...] = c1

    def attention(tp, h1val):
        fm = fmap_v[...]
        h1b = h1val.astype(jnp.bfloat16)
        att_v[...] = jnp.sum(fm * h1b[:, None, :], axis=-1,
                             dtype=jnp.bfloat16)
        av = jax.nn.sigmoid(att_v[...].astype(F32))
        mask_ref[tp] = av
        avb = av.astype(jnp.bfloat16)
        ctx = jnp.sum(fm * avb[:, :, None], axis=1,
                      dtype=F32)
        ctx_ref[tp] = ctx

    def step(t, _):
        lstm_step(t)
        attention(t, h1_v[...])
        return ()

    jax.lax.fori_loop(0, num_steps, step, (), unroll=False)
    cr = ctx_ref[...].reshape(num_steps * bc, hid)
    g_ref[...] = (jnp.dot(cr, genw_ref[...], preferred_element_type=F32)
                  + genb_ref[...]).reshape(num_steps, bc, -1)


def kernel(origin_fmap, text, emb_W, emb_b, W_ih0, W_hh0, b_ih0, b_hh0,
           W_ih1, W_hh1, b_ih1, b_hh1, gen_W, gen_b, is_train):
    b, c, h, w = origin_fmap.shape
    hid = c
    l = h * w
    t = text.shape[1]
    nc = gen_W.shape[0]
    ncp = 128
    bc = 64
    nb = b // bc

    fmap = jnp.transpose(origin_fmap, (0, 2, 3, 1)).reshape(b, l, hid)
    fmap = fmap.astype(jnp.bfloat16)
    onehot = (text.T[:, :, None] == jnp.arange(ncp)[None, None, :]).astype(F32)
    emb_pad = jnp.zeros((ncp, hid), F32).at[:nc].set(emb_W.T) + emb_b[None, :]
    w_all = jnp.stack([W_ih0.T, W_hh0.T, W_ih1.T, W_hh1.T]).astype(jnp.bfloat16)
    b0 = (b_ih0 + b_hh0)[None, :]
    b1 = (b_ih1 + b_hh1)[None, :]
    genw = jnp.zeros((hid, ncp), F32).at[:, :nc].set(gen_W.T)
    genb = jnp.zeros((1, ncp), F32).at[0, :nc].set(gen_b)

    kfn = functools.partial(_decode_kernel, bc=bc, num_steps=t, hid=hid)
    g_all, ctx_all, mask_all = pl.pallas_call(
        kfn,
        grid=(nb,),
        in_specs=[
            pl.BlockSpec(memory_space=pl.ANY),
            pl.BlockSpec(memory_space=pl.ANY),
            pl.BlockSpec((t, bc, ncp), lambda i: (0, i, 0)),
            pl.BlockSpec((ncp, hid), lambda i: (0, 0)),
            pl.BlockSpec((1, 4 * hid), lambda i: (0, 0)),
            pl.BlockSpec((1, 4 * hid), lambda i: (0, 0)),
            pl.BlockSpec((hid, ncp), lambda i: (0, 0)),
            pl.BlockSpec((1, ncp), lambda i: (0, 0)),
        ],
        out_specs=[
            pl.BlockSpec((t, bc, ncp), lambda i: (0, i, 0)),
            pl.BlockSpec((t, bc, hid), lambda i: (0, i, 0)),
            pl.BlockSpec((t, bc, l), lambda i: (0, i, 0)),
        ],
        out_shape=[
            jax.ShapeDtypeStruct((t, b, ncp), F32),
            jax.ShapeDtypeStruct((t, b, hid), F32),
            jax.ShapeDtypeStruct((t, b, l), F32),
        ],
        scratch_shapes=[
            pltpu.VMEM((bc, l, hid), jnp.bfloat16),
            pltpu.VMEM((4, hid, 4 * hid), jnp.bfloat16),
            pltpu.VMEM((bc, hid), F32),
            pltpu.VMEM((bc, hid), F32),
            pltpu.VMEM((bc, hid), F32),
            pltpu.VMEM((bc, hid), F32),
            pltpu.VMEM((bc, l), jnp.bfloat16),
            pltpu.SemaphoreType.DMA,
            pltpu.SemaphoreType.DMA,
        ],
        compiler_params=pltpu.CompilerParams(
            dimension_semantics=("parallel",),
            vmem_limit_bytes=56 * 1024 * 1024,
        ),
        name="hybrid_branch_decode",
    )(fmap, w_all, onehot, emb_pad, b0, b1, genw, genb)

    g = jnp.transpose(g_all, (1, 0, 2))[:, :, :nc]
    output_hiddens = jnp.transpose(ctx_all, (1, 0, 2))
    masks = jnp.transpose(mask_all, (1, 0, 2)).reshape(b, t, h, w)
    return g, output_hiddens, masks


# speedup vs baseline: 1.9417x; 1.0789x over previous
import functools

import jax
import jax.numpy as jnp
from jax.experimental import pallas as pl
from jax.experimental.pallas import tpu as pltpu

F32 = jnp.float32


def _decode_kernel(
    fmap_any,
    w_any,
    onehot_ref,
    emb_ref,
    b0_ref,
    b1_ref,
    genw_ref,
    genb_ref,
    g_ref,
    ctx_ref,
    mask_ref,
    fmap_v,
    w_v,
    h0_v, c0_v, h1_v, c1_v,
    att_v,
    sem0, sem1,
    *, bc, num_steps, hid,
):
    i = pl.program_id(0)
    cp_f = pltpu.make_async_copy(fmap_any.at[pl.ds(i * bc, bc)], fmap_v, sem0)
    cp_f.start()
    cp_w = pltpu.make_async_copy(w_any, w_v, sem1)
    cp_w.start()

    zeros = jnp.zeros((bc, hid), F32)
    h0_v[...] = zeros
    c0_v[...] = zeros
    h1_v[...] = zeros
    c1_v[...] = zeros

    cp_w.wait()
    cp_f.wait()

    bf = jnp.bfloat16

    def lstm(x, h, c, w_ih, w_hh, b):
        gates = (jnp.dot(x.astype(bf), w_ih, preferred_element_type=F32)
                 + jnp.dot(h.astype(bf), w_hh, preferred_element_type=F32) + b)
        gi = jax.nn.sigmoid(gates[:, 0:hid])
        gf = jax.nn.sigmoid(gates[:, hid:2 * hid])
        gg = jnp.tanh(gates[:, 2 * hid:3 * hid])
        go = jax.nn.sigmoid(gates[:, 3 * hid:4 * hid])
        c_new = gf * c + gi * gg
        h_new = go * jnp.tanh(c_new)
        return h_new, c_new

    def lstm_step(t):
        oh = onehot_ref[t]
        x = jnp.dot(oh, emb_ref[...], preferred_element_type=F32)
        h0, c0 = lstm(x, h0_v[...], c0_v[...], w_v[0], w_v[1], b0_ref[...])
        h0_v[...] = h0
        c0_v[...] = c0
        h1, c1 = lstm(h0, h1_v[...], c1_v[...], w_v[2], w_v[3], b1_ref[...])
        h1_v[...] = h1
        c1_v[...] = c1

    def attention(tp, h1val):
        fm = fmap_v[...]
        h1b = h1val.astype(jnp.bfloat16)
        att_v[...] = jnp.sum(fm * h1b[:, None, :], axis=-1,
                             dtype=jnp.bfloat16)
        av = jax.nn.sigmoid(att_v[...].astype(F32))
        mask_ref[tp] = av
        avb = av.astype(jnp.bfloat16)
        prod = fm * avb[:, :, None]
        quar = jnp.sum(prod.reshape(prod.shape[0], 4, prod.shape[1] // 4,
                                    prod.shape[2]),
                       axis=1, dtype=jnp.bfloat16)
        ctx = jnp.sum(quar, axis=1, dtype=F32)
        ctx_ref[tp] = ctx

    def step(t, _):
        lstm_step(t)
        attention(t, h1_v[...])
        return ()

    jax.lax.fori_loop(0, num_steps, step, (), unroll=False)
    cr = ctx_ref[...].reshape(num_steps * bc, hid)
    g_ref[...] = (jnp.dot(cr, genw_ref[...], preferred_element_type=F32)
                  + genb_ref[...]).reshape(num_steps, bc, -1)


def kernel(origin_fmap, text, emb_W, emb_b, W_ih0, W_hh0, b_ih0, b_hh0,
           W_ih1, W_hh1, b_ih1, b_hh1, gen_W, gen_b, is_train):
    b, c, h, w = origin_fmap.shape
    hid = c
    l = h * w
    t = text.shape[1]
    nc = gen_W.shape[0]
    ncp = 128
    bc = 64
    nb = b // bc

    fmap = jnp.transpose(origin_fmap, (0, 2, 3, 1)).reshape(b, l, hid)
    fmap = fmap.astype(jnp.bfloat16)
    onehot = (text.T[:, :, None] == jnp.arange(ncp)[None, None, :]).astype(F32)
    emb_pad = jnp.zeros((ncp, hid), F32).at[:nc].set(emb_W.T) + emb_b[None, :]
    w_all = jnp.stack([W_ih0.T, W_hh0.T, W_ih1.T, W_hh1.T]).astype(jnp.bfloat16)
    b0 = (b_ih0 + b_hh0)[None, :]
    b1 = (b_ih1 + b_hh1)[None, :]
    genw = jnp.zeros((hid, ncp), F32).at[:, :nc].set(gen_W.T)
    genb = jnp.zeros((1, ncp), F32).at[0, :nc].set(gen_b)

    kfn = functools.partial(_decode_kernel, bc=bc, num_steps=t, hid=hid)
    g_all, ctx_all, mask_all = pl.pallas_call(
        kfn,
        grid=(nb,),
        in_specs=[
            pl.BlockSpec(memory_space=pl.ANY),
            pl.BlockSpec(memory_space=pl.ANY),
            pl.BlockSpec((t, bc, ncp), lambda i: (0, i, 0)),
            pl.BlockSpec((ncp, hid), lambda i: (0, 0)),
            pl.BlockSpec((1, 4 * hid), lambda i: (0, 0)),
            pl.BlockSpec((1, 4 * hid), lambda i: (0, 0)),
            pl.BlockSpec((hid, ncp), lambda i: (0, 0)),
            pl.BlockSpec((1, ncp), lambda i: (0, 0)),
        ],
        out_specs=[
            pl.BlockSpec((t, bc, ncp), lambda i: (0, i, 0)),
            pl.BlockSpec((t, bc, hid), lambda i: (0, i, 0)),
            pl.BlockSpec((t, bc, l), lambda i: (0, i, 0)),
        ],
        out_shape=[
            jax.ShapeDtypeStruct((t, b, ncp), F32),
            jax.ShapeDtypeStruct((t, b, hid), F32),
            jax.ShapeDtypeStruct((t, b, l), F32),
        ],
        scratch_shapes=[
            pltpu.VMEM((bc, l, hid), jnp.bfloat16),
            pltpu.VMEM((4, hid, 4 * hid), jnp.bfloat16),
            pltpu.VMEM((bc, hid), F32),
            pltpu.VMEM((bc, hid), F32),
            pltpu.VMEM((bc, hid), F32),
            pltpu.VMEM((bc, hid), F32),
            pltpu.VMEM((bc, l), jnp.bfloat16),
            pltpu.SemaphoreType.DMA,
            pltpu.SemaphoreType.DMA,
        ],
        compiler_params=pltpu.CompilerParams(
            dimension_semantics=("parallel",),
            vmem_limit_bytes=56 * 1024 * 1024,
        ),
        name="hybrid_branch_decode",
    )(fmap, w_all, onehot, emb_pad, b0, b1, genw, genb)

    g = jnp.transpose(g_all, (1, 0, 2))[:, :, :nc]
    output_hiddens = jnp.transpose(ctx_all, (1, 0, 2))
    masks = jnp.transpose(mask_all, (1, 0, 2)).reshape(b, t, h, w)
    return g, output_hiddens, masks


# speedup vs baseline: 2.2092x; 1.1378x over previous
import functools

import jax
import jax.numpy as jnp
from jax.experimental import pallas as pl
from jax.experimental.pallas import tpu as pltpu

F32 = jnp.float32
BF16 = jnp.bfloat16


def _decode_kernel(
    fmap_any,
    w_any,
    onehot_ref,
    emb_ref,
    b0_ref,
    b1_ref,
    genw_ref,
    genb_ref,
    g_any,
    ctx_any,
    mask_any,
    fmap_v,
    w_v,
    h0_v, c0_v, h1_v, c1_v,
    att_v,
    g_s,
    ctx_s,
    mask_s,
    sem0, sem1, sem2, sem3, sem4,
    *, bc, num_steps, hid,
):
    i = pl.program_id(0)
    cp_f = pltpu.make_async_copy(fmap_any.at[pl.ds(i * bc, bc)], fmap_v, sem0)
    cp_f.start()
    cp_w = pltpu.make_async_copy(w_any, w_v, sem1)
    cp_w.start()

    zeros = jnp.zeros((bc, hid), F32)
    h0_v[...] = zeros
    c0_v[...] = zeros
    h1_v[...] = zeros
    c1_v[...] = zeros

    cp_w.wait()
    cp_f.wait()

    def lstm(x, h, c, w_ih, w_hh, b):
        gates = (jnp.dot(x, w_ih, preferred_element_type=F32)
                 + jnp.dot(h.astype(BF16), w_hh, preferred_element_type=F32)
                 + b)
        gi = jax.nn.sigmoid(gates[:, 0:hid])
        gf = jax.nn.sigmoid(gates[:, hid:2 * hid])
        gg = jnp.tanh(gates[:, 2 * hid:3 * hid])
        go = jax.nn.sigmoid(gates[:, 3 * hid:4 * hid])
        c_new = gf * c + gi * gg
        h_new = go * jnp.tanh(c_new)
        return h_new, c_new

    def lstm_step(t):
        oh = onehot_ref[t]
        x = jnp.dot(oh, emb_ref[...], preferred_element_type=F32)
        h0, c0 = lstm(x.astype(BF16), h0_v[...], c0_v[...],
                      w_v[0], w_v[1], b0_ref[...])
        h0_v[...] = h0
        c0_v[...] = c0
        h1, c1 = lstm(h0.astype(BF16), h1_v[...], c1_v[...],
                      w_v[2], w_v[3], b1_ref[...])
        h1_v[...] = h1
        c1_v[...] = c1

    def attention(tp, h1val):
        fm = fmap_v[...]
        h1b = h1val.astype(BF16)
        att_v[...] = jnp.sum(fm * h1b[:, None, :], axis=-1,
                             dtype=BF16)
        av = jax.nn.sigmoid(att_v[...].astype(F32))
        avb = av.astype(BF16)
        mask_s[tp] = avb
        prod = fm * avb[:, :, None]
        quar = jnp.sum(prod.reshape(prod.shape[0], 4, prod.shape[1] // 4,
                                    prod.shape[2]),
                       axis=1, dtype=BF16)
        ctx = jnp.sum(quar, axis=1, dtype=F32)
        ctx_s[tp] = ctx

    def step(t, _):
        lstm_step(t)
        attention(t, h1_v[...])
        return ()

    jax.lax.fori_loop(0, num_steps, step, (), unroll=False)
    cr = ctx_s[...].reshape(num_steps * bc, hid)
    g_s[...] = (jnp.dot(cr, genw_ref[...], preferred_element_type=F32)
                + genb_ref[...]).reshape(num_steps, bc, -1)

    sl = pl.ds(i * bc, bc)
    cp_g = pltpu.make_async_copy(g_s, g_any.at[:, sl, :], sem2)
    cp_c = pltpu.make_async_copy(ctx_s, ctx_any.at[:, sl, :], sem3)
    cp_m = pltpu.make_async_copy(mask_s, mask_any.at[:, sl, :], sem4)
    cp_g.start()
    cp_c.start()
    cp_m.start()
    cp_g.wait()
    cp_c.wait()
    cp_m.wait()


def kernel(origin_fmap, text, emb_W, emb_b, W_ih0, W_hh0, b_ih0, b_hh0,
           W_ih1, W_hh1, b_ih1, b_hh1, gen_W, gen_b, is_train):
    b, c, h, w = origin_fmap.shape
    hid = c
    l = h * w
    t = text.shape[1]
    nc = gen_W.shape[0]
    ncp = 128
    bc = 128
    nb = b // bc

    fmap = jnp.transpose(origin_fmap, (0, 2, 3, 1)).reshape(b, l, hid)
    fmap = fmap.astype(BF16)
    onehot = (text.T[:, :, None] == jnp.arange(ncp)[None, None, :]).astype(BF16)
    emb_pad = (jnp.zeros((ncp, hid), F32).at[:nc].set(emb_W.T)
               + emb_b[None, :]).astype(BF16)
    w_all = jnp.stack([W_ih0.T, W_hh0.T, W_ih1.T, W_hh1.T]).astype(BF16)
    b0 = (b_ih0 + b_hh0)[None, :]
    b1 = (b_ih1 + b_hh1)[None, :]
    genw = jnp.zeros((hid, ncp), F32).at[:, :nc].set(gen_W.T)
    genb = jnp.zeros((1, ncp), F32).at[0, :nc].set(gen_b)

    kfn = functools.partial(_decode_kernel, bc=bc, num_steps=t, hid=hid)
    g_all, ctx_all, mask_all = pl.pallas_call(
        kfn,
        grid=(nb,),
        in_specs=[
            pl.BlockSpec(memory_space=pl.ANY),
            pl.BlockSpec(memory_space=pl.ANY),
            pl.BlockSpec((t, bc, ncp), lambda i: (0, i, 0)),
            pl.BlockSpec((ncp, hid), lambda i: (0, 0)),
            pl.BlockSpec((1, 4 * hid), lambda i: (0, 0)),
            pl.BlockSpec((1, 4 * hid), lambda i: (0, 0)),
            pl.BlockSpec((hid, ncp), lambda i: (0, 0)),
            pl.BlockSpec((1, ncp), lambda i: (0, 0)),
        ],
        out_specs=[
            pl.BlockSpec(memory_space=pl.ANY),
            pl.BlockSpec(memory_space=pl.ANY),
            pl.BlockSpec(memory_space=pl.ANY),
        ],
        out_shape=[
            jax.ShapeDtypeStruct((t, b, ncp), F32),
            jax.ShapeDtypeStruct((t, b, hid), F32),
            jax.ShapeDtypeStruct((t, b, l), BF16),
        ],
        scratch_shapes=[
            pltpu.VMEM((bc, l, hid), BF16),
            pltpu.VMEM((4, hid, 4 * hid), BF16),
            pltpu.VMEM((bc, hid), F32),
            pltpu.VMEM((bc, hid), F32),
            pltpu.VMEM((bc, hid), F32),
            pltpu.VMEM((bc, hid), F32),
            pltpu.VMEM((bc, l), BF16),
            pltpu.VMEM((t, bc, ncp), F32),
            pltpu.VMEM((t, bc, hid), F32),
            pltpu.VMEM((t, bc, l), BF16),
            pltpu.SemaphoreType.DMA,
            pltpu.SemaphoreType.DMA,
            pltpu.SemaphoreType.DMA,
            pltpu.SemaphoreType.DMA,
            pltpu.SemaphoreType.DMA,
        ],
        compiler_params=pltpu.CompilerParams(
            dimension_semantics=("parallel",),
            vmem_limit_bytes=58 * 1024 * 1024,
        ),
        name="hybrid_branch_decode",
    )(fmap, w_all, onehot, emb_pad, b0, b1, genw, genb)

    g = jnp.transpose(g_all, (1, 0, 2))[:, :, :nc]
    output_hiddens = jnp.transpose(ctx_all, (1, 0, 2))
    masks = jnp.transpose(mask_all.astype(F32), (1, 0, 2)).reshape(b, t, h, w)
    return g, output_hiddens, masks
